# baseline (speedup 1.0000x reference)
import functools

import jax
import jax.numpy as jnp
from jax import lax
from jax.experimental import pallas as pl
from jax.experimental.pallas import tpu as pltpu
from jax.experimental.pallas import tpu_sc as plsc

H = 64
MEM = 8
SEQ = 64
VOCAB = 66
NOUT = 64
B = 16384
BT = 2048
NW = 32
BSC = 8192
BPW = BSC // NW


def _fold_body(emb_ref, w1_ref, b1_ref, t2_ref):
    emb = emb_ref[...]
    w1 = w1_ref[...]
    dn = (((1,), (1,)), ((), ()))
    t2_ref[:, :H] = jax.lax.dot_general(
        emb, w1[:, :H], dn, preferred_element_type=jnp.float32) + b1_ref[...]
    t2_ref[:, H:] = jax.lax.dot_general(
        emb, w1[:, H:], dn, preferred_element_type=jnp.float32) * (1.0 / MEM)


def _fold(embed, W1, b1):
    return pl.pallas_call(
        _fold_body,
        out_shape=jax.ShapeDtypeStruct((VOCAB, 2 * H), jnp.float32),
    )(embed, W1, b1)


def _make_sc_gather():
    mesh = plsc.VectorSubcoreMesh(core_axis_name="c", subcore_axis_name="s")

    @functools.partial(
        pl.kernel, mesh=mesh,
        out_type=jax.ShapeDtypeStruct((BSC, H), jnp.float32),
        scratch_types=[
            pltpu.VMEM((VOCAB * 2 * H,), jnp.float32),
            pltpu.VMEM((BPW // 8, 128), jnp.int32),
            pltpu.VMEM((BPW, H), jnp.float32),
        ],
    )
    def sc_gather(t2_hbm, idx_hbm, out_hbm, t2_v, idx_v, acc_v):
        wid = lax.axis_index("s") * 2 + lax.axis_index("c")
        base = wid * BPW
        pltpu.sync_copy(t2_hbm, t2_v)
        row0 = pl.multiple_of(base // 8, 8)
        pltpu.sync_copy(idx_hbm.at[pl.ds(row0, BPW // 8)], idx_v)

        def accum_row(r, tv):
            offs = [tv[j] for j in range(9)]
            for c in range(4):
                ls = [t2_v[pl.ds(offs[0] + c * 16, 16)]]
                ls += [t2_v[pl.ds(offs[j] + (H + c * 16), 16)]
                       for j in range(1, 9)]
                s0 = (ls[0] + ls[1]) + (ls[2] + ls[3])
                s1 = (ls[4] + ls[5]) + (ls[6] + ls[7])
                acc_v[r, pl.ds(c * 16, 16)] = (s0 + s1) + ls[8]

        @plsc.parallel_loop(0, BPW // 2, unroll=2)
        def body(g):
            tv0 = idx_v[g // 4, pl.ds((g % 4) * 32, 16)] * (2 * H)
            tv1 = idx_v[g // 4, pl.ds((g % 4) * 32 + 16, 16)] * (2 * H)
            accum_row(2 * g, tv0)
            accum_row(2 * g + 1, tv1)

        out0 = pl.multiple_of(base, 8)
        pltpu.sync_copy(acc_v, out_hbm.at[pl.ds(out0, BPW)])

    return sc_gather


_sc_gather = _make_sc_gather()


def _mlp_body(buf_ref, pre_ref, w2_ref, b2_ref, out_ref):
    del buf_ref
    h = jnp.maximum(pre_ref[...], 0.0)
    dn = (((1,), (1,)), ((), ()))
    out_ref[...] = jax.lax.dot_general(
        h, w2_ref[...], dn, preferred_element_type=jnp.float32) + b2_ref[...]


def _mlp(buf, pre, W2, b2):
    return pl.pallas_call(
        _mlp_body,
        grid=(BSC // BT,),
        in_specs=[
            pl.BlockSpec(memory_space=pl.ANY),
            pl.BlockSpec((BT, H), lambda i: (i, 0)),
            pl.BlockSpec((NOUT, H), lambda i: (0, 0)),
            pl.BlockSpec((1, NOUT), lambda i: (0, 0)),
        ],
        out_specs=pl.BlockSpec((BT, NOUT), lambda i: (i, 0)),
        out_shape=jax.ShapeDtypeStruct((B, NOUT), jnp.float32),
        input_output_aliases={0: 0},
    )(buf, pre, W2, b2)


def _fused_body(idx_ref, emb_ref, w1_ref, b1_ref, w2_ref, b2_ref, out_ref):
    emb = emb_ref[...]
    w1 = w1_ref[...]
    dn = (((1,), (1,)), ((), ()))
    eq = jax.lax.dot_general(emb, w1[:, :H], dn,
                             preferred_element_type=jnp.float32) + b1_ref[...]
    em = jax.lax.dot_general(emb, w1[:, H:], dn,
                             preferred_element_type=jnp.float32) * (1.0 / MEM)

    idx = idx_ref[...].astype(jnp.int32)
    iota = jax.lax.broadcasted_iota(jnp.int32, (1, VOCAB), 1)
    q1 = (idx[:, 0:1] == iota).astype(jnp.float32)
    cnt = (idx[:, 1:2] == iota).astype(jnp.float32)
    for j in range(2, MEM + 1):
        cnt += (idx[:, j:j + 1] == iota).astype(jnp.float32)

    pre = (jnp.dot(q1, eq, preferred_element_type=jnp.float32)
           + jnp.dot(cnt, em, preferred_element_type=jnp.float32))
    h = jnp.maximum(pre, 0.0)
    out_ref[...] = jax.lax.dot_general(h, w2_ref[...], dn,
                                       preferred_element_type=jnp.float32) + b2_ref[...]


FBT = 1024


def _fused(idx8, embed, W1, b1, W2, b2):
    nrows = B - BSC
    blk0 = BSC // FBT
    return pl.pallas_call(
        _fused_body,
        grid=(nrows // FBT,),
        in_specs=[
            pl.BlockSpec((FBT, 16), lambda i: (i + blk0, 0)),
            pl.BlockSpec((VOCAB, H), lambda i: (0, 0)),
            pl.BlockSpec((H, 2 * H), lambda i: (0, 0)),
            pl.BlockSpec((1, H), lambda i: (0, 0)),
            pl.BlockSpec((NOUT, H), lambda i: (0, 0)),
            pl.BlockSpec((1, NOUT), lambda i: (0, 0)),
        ],
        out_specs=pl.BlockSpec((FBT, NOUT), lambda i: (i + blk0, 0)),
        out_shape=jax.ShapeDtypeStruct((B, NOUT), jnp.float32),
    )(idx8, embed, W1, b1, W2, b2)


@jax.jit
def _run(idx8_16, idx8_pad, embed, W1, b1, W2, b2):
    t2 = _fold(embed, W1, b1).reshape(-1)
    pre = _sc_gather(t2, idx8_pad)
    buf = _fused(idx8_16, embed, W1, b1, W2, b2)
    return _mlp(buf, pre, W2, b2)


def kernel(seqs, query_tok, embed, W1, b1, W2, b2):
    start = SEQ - 1 - MEM
    idx16 = jnp.concatenate(
        [query_tok[:, None], seqs[:, start:SEQ - 1],
         jnp.zeros((B, 16 - (MEM + 1)), jnp.int32)],
        axis=1).astype(jnp.int32)
    idx2d = idx16.reshape(B // 8, 128)
    return _run(idx16, idx2d, embed, W1,
                b1.reshape(1, H), W2, b2.reshape(1, NOUT))


# speedup vs baseline: 1.0281x; 1.0281x over previous
import functools

import jax
import jax.numpy as jnp
from jax import lax
from jax.experimental import pallas as pl
from jax.experimental.pallas import tpu as pltpu
from jax.experimental.pallas import tpu_sc as plsc

H = 64
MEM = 8
SEQ = 64
VOCAB = 66
NOUT = 64
B = 16384
BT = 2048
NW = 32
BSC = 8192
BPW = BSC // NW


def _fold_body(emb_ref, w1_ref, b1_ref, t2_ref):
    emb = emb_ref[...]
    w1 = w1_ref[...]
    dn = (((1,), (1,)), ((), ()))
    t2_ref[:, :H] = jax.lax.dot_general(
        emb, w1[:, :H], dn, preferred_element_type=jnp.float32) + b1_ref[...]
    t2_ref[:, H:] = jax.lax.dot_general(
        emb, w1[:, H:], dn, preferred_element_type=jnp.float32) * (1.0 / MEM)


def _fold(embed, W1, b1):
    return pl.pallas_call(
        _fold_body,
        out_shape=jax.ShapeDtypeStruct((VOCAB, 2 * H), jnp.float32),
    )(embed, W1, b1)


def _make_sc_gather():
    mesh = plsc.VectorSubcoreMesh(core_axis_name="c", subcore_axis_name="s")

    @functools.partial(
        pl.kernel, mesh=mesh,
        out_type=jax.ShapeDtypeStruct((BSC, H), jnp.float32),
        scratch_types=[
            pltpu.VMEM((VOCAB * 2 * H,), jnp.float32),
            pltpu.VMEM((BPW // 8, 128), jnp.int32),
            pltpu.VMEM((BPW, H), jnp.float32),
        ],
    )
    def sc_gather(t2_hbm, idx_hbm, out_hbm, t2_v, idx_v, acc_v):
        wid = lax.axis_index("s") * 2 + lax.axis_index("c")
        base = wid * BPW
        pltpu.sync_copy(t2_hbm, t2_v)
        row0 = pl.multiple_of(base // 8, 8)
        pltpu.sync_copy(idx_hbm.at[pl.ds(row0, BPW // 8)], idx_v)

        def accum_row(r, tv):
            offs = [tv[j] for j in range(9)]
            for c in range(4):
                ls = [t2_v[pl.ds(offs[0] + c * 16, 16)]]
                ls += [t2_v[pl.ds(offs[j] + (H + c * 16), 16)]
                       for j in range(1, 9)]
                s0 = (ls[0] + ls[1]) + (ls[2] + ls[3])
                s1 = (ls[4] + ls[5]) + (ls[6] + ls[7])
                acc_v[r, pl.ds(c * 16, 16)] = (s0 + s1) + ls[8]

        @plsc.parallel_loop(0, BPW // 2, unroll=2)
        def body(g):
            tv0 = idx_v[g // 4, pl.ds((g % 4) * 32, 16)] * (2 * H)
            tv1 = idx_v[g // 4, pl.ds((g % 4) * 32 + 16, 16)] * (2 * H)
            accum_row(2 * g, tv0)
            accum_row(2 * g + 1, tv1)

        out0 = pl.multiple_of(base, 8)
        pltpu.sync_copy(acc_v, out_hbm.at[pl.ds(out0, BPW)])

    return sc_gather


_sc_gather = _make_sc_gather()


def _mlp_body(buf_ref, pre_ref, w2_ref, b2_ref, out_ref):
    del buf_ref
    h = jnp.maximum(pre_ref[...], 0.0)
    dn = (((1,), (1,)), ((), ()))
    out_ref[...] = jax.lax.dot_general(
        h, w2_ref[...], dn, preferred_element_type=jnp.float32) + b2_ref[...]


def _mlp(buf, pre, W2, b2):
    return pl.pallas_call(
        _mlp_body,
        grid=(BSC // BT,),
        in_specs=[
            pl.BlockSpec(memory_space=pl.ANY),
            pl.BlockSpec((BT, H), lambda i: (i, 0)),
            pl.BlockSpec((NOUT, H), lambda i: (0, 0)),
            pl.BlockSpec((1, NOUT), lambda i: (0, 0)),
        ],
        out_specs=pl.BlockSpec((BT, NOUT), lambda i: (i, 0)),
        out_shape=jax.ShapeDtypeStruct((B, NOUT), jnp.float32),
        input_output_aliases={0: 0},
    )(buf, pre, W2, b2)


def _fused_body(idx_ref, emb_ref, w1_ref, b1_ref, w2_ref, b2_ref, out_ref):
    emb = emb_ref[...]
    w1 = w1_ref[...]
    dn = (((1,), (1,)), ((), ()))
    eq = jax.lax.dot_general(emb, w1[:, :H], dn,
                             preferred_element_type=jnp.float32) + b1_ref[...]
    em = jax.lax.dot_general(emb, w1[:, H:], dn,
                             preferred_element_type=jnp.float32) * (1.0 / MEM)

    idx = idx_ref[...].astype(jnp.int32)
    iota = jax.lax.broadcasted_iota(jnp.int32, (1, VOCAB), 1)
    q1 = (idx[:, 0:1] == iota).astype(jnp.float32)
    cnt = (idx[:, 1:2] == iota).astype(jnp.float32)
    for j in range(2, MEM + 1):
        cnt += (idx[:, j:j + 1] == iota).astype(jnp.float32)

    pre = (jnp.dot(q1, eq, preferred_element_type=jnp.float32)
           + jnp.dot(cnt, em, preferred_element_type=jnp.float32))
    h = jnp.maximum(pre, 0.0)
    out_ref[...] = jax.lax.dot_general(h, w2_ref[...], dn,
                                       preferred_element_type=jnp.float32) + b2_ref[...]


FBT = 2048


def _fused(idx8, embed, W1, b1, W2, b2):
    nrows = B - BSC
    blk0 = BSC // FBT
    return pl.pallas_call(
        _fused_body,
        grid=(nrows // FBT,),
        in_specs=[
            pl.BlockSpec((FBT, 16), lambda i: (i + blk0, 0)),
            pl.BlockSpec((VOCAB, H), lambda i: (0, 0)),
            pl.BlockSpec((H, 2 * H), lambda i: (0, 0)),
            pl.BlockSpec((1, H), lambda i: (0, 0)),
            pl.BlockSpec((NOUT, H), lambda i: (0, 0)),
            pl.BlockSpec((1, NOUT), lambda i: (0, 0)),
        ],
        out_specs=pl.BlockSpec((FBT, NOUT), lambda i: (i + blk0, 0)),
        out_shape=jax.ShapeDtypeStruct((B, NOUT), jnp.float32),
    )(idx8, embed, W1, b1, W2, b2)


@jax.jit
def _run(idx8_16, idx8_pad, embed, W1, b1, W2, b2):
    t2 = _fold(embed, W1, b1).reshape(-1)
    pre = _sc_gather(t2, idx8_pad)
    buf = _fused(idx8_16, embed, W1, b1, W2, b2)
    return _mlp(buf, pre, W2, b2)


def kernel(seqs, query_tok, embed, W1, b1, W2, b2):
    start = SEQ - 1 - MEM
    idx16 = jnp.concatenate(
        [query_tok[:, None], seqs[:, start:SEQ - 1],
         jnp.zeros((B, 16 - (MEM + 1)), jnp.int32)],
        axis=1).astype(jnp.int32)
    idx2d = idx16.reshape(B // 8, 128)
    return _run(idx16, idx2d, embed, W1,
                b1.reshape(1, H), W2, b2.reshape(1, NOUT))
